# baseline (speedup 1.0000x reference)
import functools

import jax
import jax.numpy as jnp
from jax import lax
from jax.experimental import pallas as pl
from jax.experimental.pallas import tpu as pltpu
from jax.experimental.pallas import tpu_sc as plsc

NC, NS, L = 2, 16, 16
K = 128
KD = 80
U = 8
N_PAD = 10240
RP = N_PAD // NS
ZR = 64
DEGW = 16

_f32 = jnp.float32


def _mesh():
    return plsc.VectorSubcoreMesh(core_axis_name="c", subcore_axis_name="s")


def _sc_degree(C, K):
    @functools.partial(
        pl.kernel,
        out_type=jax.ShapeDtypeStruct((NC, N_PAD, DEGW), _f32),
        mesh=_mesh(),
        scratch_types=[
            pltpu.VMEM((C, K), jnp.int32),
            pltpu.VMEM((K, DEGW), _f32),
            pltpu.VMEM((ZR, DEGW), _f32),
            pltpu.VMEM_SHARED((N_PAD, DEGW), _f32),
            pltpu.SemaphoreType.DMA,
        ],
        compiler_params=pltpu.CompilerParams(use_tc_tiling_on_sc=False),
    )
    def deg_kernel(edges_hbm, out_hbm, dst_v, ones_v, zb_v, acc_sh, ssem):
        c = lax.axis_index("c")
        s = lax.axis_index("s")
        pltpu.sync_copy(edges_hbm.at[1, c, s], dst_v)

        def fill_ones(i, _):
            ones_v[i, :] = jnp.ones((L,), _f32)
            return 0

        lax.fori_loop(0, K, fill_ones, 0)

        def fill_zero(i, _):
            zb_v[i, :] = jnp.zeros((L,), _f32)
            return 0

        lax.fori_loop(0, ZR, fill_zero, 0)
        for t in range(RP // ZR):
            pltpu.sync_copy(zb_v, acc_sh.at[pl.ds(s * RP + t * ZR, ZR)])
        plsc.subcore_barrier()

        def body(i, _):
            pltpu.async_copy(ones_v, acc_sh.at[dst_v.at[i]], ssem, add=True)
            return 0

        lax.fori_loop(0, C, body, 0)

        def drain(i, _):
            pltpu.make_async_copy(ones_v, acc_sh.at[dst_v.at[i]], ssem).wait()
            return 0

        lax.fori_loop(0, C, drain, 0)
        plsc.subcore_barrier()
        pltpu.sync_copy(acc_sh.at[pl.ds(s * RP, RP)],
                        out_hbm.at[c, pl.ds(s * RP, RP)])

    return deg_kernel


def _sc_aggregate(C, F, NROWS):
    _bf = jnp.bfloat16

    @functools.partial(
        pl.kernel,
        out_type=jax.ShapeDtypeStruct((NC, N_PAD, F), _bf),
        mesh=_mesh(),
        scratch_types=[
            pltpu.VMEM((C, K), jnp.int32),
            pltpu.VMEM((C, K), jnp.int32),
            pltpu.VMEM((K, F), _bf),
            pltpu.VMEM((ZR, F), _bf),
            pltpu.VMEM_SHARED((N_PAD, F), _bf),
            pltpu.VMEM_SHARED((NROWS, F), _bf),
            pltpu.SemaphoreType.DMA,
        ],
        compiler_params=pltpu.CompilerParams(use_tc_tiling_on_sc=False),
    )
    def agg_kernel(g_hbm, src_hbm, dst_hbm, out_hbm,
                   src_v, dst_v, rows_v, zb_v, acc_sh, g_sh, sem):
        c = lax.axis_index("c")
        s = lax.axis_index("s")
        pltpu.sync_copy(src_hbm.at[c, s], src_v)
        pltpu.sync_copy(dst_hbm.at[c, s], dst_v)
        gr = NROWS // NS
        pltpu.sync_copy(g_hbm.at[pl.ds(s * gr, gr)], g_sh.at[pl.ds(s * gr, gr)])

        def fill_zero(i, _):
            for j in range(F // (2 * L)):
                zb_v[i, pl.ds(j * 2 * L, 2 * L)] = jnp.zeros((2 * L,), _bf)
            return 0

        lax.fori_loop(0, ZR, fill_zero, 0)
        for t in range(RP // ZR):
            pltpu.sync_copy(zb_v, acc_sh.at[pl.ds(s * RP + t * ZR, ZR)])
        plsc.subcore_barrier()

        def body(i, _):
            pltpu.async_copy(g_sh.at[src_v.at[i]], rows_v, sem).wait()
            pltpu.sync_copy(rows_v, acc_sh.at[dst_v.at[i]], add=True)
            return 0

        lax.fori_loop(0, C, body, 0)
        plsc.subcore_barrier()
        pltpu.sync_copy(acc_sh.at[pl.ds(s * RP, RP)],
                        out_hbm.at[c, pl.ds(s * RP, RP)])

    return agg_kernel


BN = 1000


def _rows(width, dtype=None):
    return pl.BlockSpec((BN, width), lambda i: (i, 0))


def _rows3(width):
    return pl.BlockSpec((NC, BN, width), lambda i: (0, i, 0))


def _full(shape):
    nd = len(shape)
    return pl.BlockSpec(shape, (lambda i: (0,) * nd))


def _tc0(x, W1, z_ref):
    z_ref[...] = jnp.dot(x[...], W1[...], preferred_element_type=_f32)


def _dv(dp):
    return lax.rsqrt(dp[0, :, 0:1] + dp[1, :, 0:1] + 1.0)


def _tc1(dp, z, g1_ref, g1b_ref):
    G = _dv(dp) * z[...]
    g1_ref[...] = G
    g1b_ref[...] = G.astype(jnp.bfloat16)


def _tc2(dp, a, g1, W2, b1, g2_ref, g2b_ref):
    dv = _dv(dp)
    agg = a[0].astype(_f32) + a[1].astype(_f32)
    H1 = jnp.maximum(dv * (agg + g1[...]) + b1[...][None, :], 0.0)
    G = dv * jnp.dot(H1, W2[...], preferred_element_type=_f32)
    g2_ref[...] = G
    g2b_ref[...] = G.astype(jnp.bfloat16)


def _tc3(dp, a, g2, b2, g3_ref, g3b_ref):
    dv = _dv(dp)
    agg = a[0].astype(_f32) + a[1].astype(_f32)
    H2 = jnp.maximum(dv * (agg + g2[...]) + b2[...][None, :], 0.0)
    G = dv * H2
    g3_ref[...] = G
    g3b_ref[...] = G.astype(jnp.bfloat16)


def _tc4(dp, a, g3, W3, b3, out_ref):
    dv = _dv(dp)
    agg = a[0].astype(_f32) + a[1].astype(_f32)
    M = dv * (agg + g3[...])
    O = jnp.dot(M, W3[...], preferred_element_type=_f32) + b3[...][None, :]
    m = jnp.max(O, axis=1, keepdims=True)
    out_ref[...] = O - m - jnp.log(jnp.sum(jnp.exp(O - m), axis=1, keepdims=True))


def _tc_call(body, n_blocks, in_specs, out_specs, shapes_dtypes, *args):
    out_shape = tuple(jax.ShapeDtypeStruct(sh, dt) for sh, dt in shapes_dtypes)
    out_specs = tuple(out_specs)
    if len(out_shape) == 1:
        out_shape, out_specs = out_shape[0], out_specs[0]
    return pl.pallas_call(
        body, grid=(n_blocks,), in_specs=in_specs, out_specs=out_specs,
        out_shape=out_shape)(*args)


def kernel(x, edge_index, W1, b1, W2, b2, W3, b3):
    N = x.shape[0]
    E = edge_index.shape[1]
    H1, H2 = W1.shape[1], W2.shape[1]

    src = edge_index[0].astype(jnp.int32)
    dst = edge_index[1].astype(jnp.int32)
    SL = NC * NS
    C = -(-E // (SL * K))
    pad = SL * C * K - E
    srcp = jnp.concatenate([src, jnp.zeros((pad,), jnp.int32)]).reshape(NC, NS, C, K)
    dstp = jnp.concatenate([dst, jnp.full((pad,), N, jnp.int32)]).reshape(NC, NS, C, K)
    if E % (SL * KD) == 0:
        CD = E // (SL * KD)
        edges5 = edge_index.astype(jnp.int32).reshape(2, NC, NS, CD, KD)
    else:
        CD = C
        edges5 = jnp.stack([srcp, dstp], axis=0)

    bf = jnp.bfloat16
    degp = _sc_degree(CD, edges5.shape[4])(edges5)

    NB = N // BN
    D_IN, D_OUT = x.shape[1], W3.shape[1]
    Z1 = _tc_call(_tc0, NB, [_rows(D_IN), _full(W1.shape)], [_rows(H1)],
                  [((N, H1), _f32)], x, W1)
    G1, G1b = _tc_call(
        _tc1, NB, [_rows3(DEGW), _rows(H1)], [_rows(H1), _rows(H1)],
        [((N, H1), _f32), ((N, H1), bf)], degp, Z1)

    a1 = _sc_aggregate(C, H1, N)(G1b, srcp, dstp)
    G2, G2b = _tc_call(
        _tc2, NB,
        [_rows3(DEGW), _rows3(H1), _rows(H1), _full(W2.shape), _full(b1.shape)],
        [_rows(H2), _rows(H2)],
        [((N, H2), _f32), ((N, H2), bf)], degp, a1, G1, W2, b1)

    a2 = _sc_aggregate(C, H2, N)(G2b, srcp, dstp)
    G3, G3b = _tc_call(
        _tc3, NB, [_rows3(DEGW), _rows3(H2), _rows(H2), _full(b2.shape)],
        [_rows(H2), _rows(H2)],
        [((N, H2), _f32), ((N, H2), bf)], degp, a2, G2, b2)

    a3 = _sc_aggregate(C, H2, N)(G3b, srcp, dstp)
    out = _tc_call(
        _tc4, NB,
        [_rows3(DEGW), _rows3(H2), _rows(H2), _full(W3.shape), _full(b3.shape)],
        [_rows(D_OUT)],
        [((N, D_OUT), _f32)], degp, a3, G3, W3, b3)
    return out


# speedup vs baseline: 1.0638x; 1.0638x over previous
import functools

import jax
import jax.numpy as jnp
from jax import lax
from jax.experimental import pallas as pl
from jax.experimental.pallas import tpu as pltpu
from jax.experimental.pallas import tpu_sc as plsc

NC, NS, L = 2, 16, 16
K = 128
KD = 80
U = 8
N_PAD = 10240
RP = N_PAD // NS
ZR = 64
DEGW = 16

_f32 = jnp.float32


def _mesh():
    return plsc.VectorSubcoreMesh(core_axis_name="c", subcore_axis_name="s")


def _sc_degree(C, K):
    @functools.partial(
        pl.kernel,
        out_type=jax.ShapeDtypeStruct((NC, N_PAD, DEGW), _f32),
        mesh=_mesh(),
        scratch_types=[
            pltpu.VMEM((C, K), jnp.int32),
            pltpu.VMEM((K, DEGW), _f32),
            pltpu.VMEM((ZR, DEGW), _f32),
            pltpu.VMEM_SHARED((N_PAD, DEGW), _f32),
            pltpu.SemaphoreType.DMA,
        ],
        compiler_params=pltpu.CompilerParams(use_tc_tiling_on_sc=False),
    )
    def deg_kernel(edges_hbm, out_hbm, dst_v, ones_v, zb_v, acc_sh, ssem):
        c = lax.axis_index("c")
        s = lax.axis_index("s")
        pltpu.sync_copy(edges_hbm.at[1, c, s], dst_v)

        def fill_ones(i, _):
            ones_v[i, :] = jnp.ones((L,), _f32)
            return 0

        lax.fori_loop(0, K, fill_ones, 0)

        def fill_zero(i, _):
            zb_v[i, :] = jnp.zeros((L,), _f32)
            return 0

        lax.fori_loop(0, ZR, fill_zero, 0)
        for t in range(RP // ZR):
            pltpu.sync_copy(zb_v, acc_sh.at[pl.ds(s * RP + t * ZR, ZR)])
        plsc.subcore_barrier()

        def body(i, _):
            pltpu.async_copy(ones_v, acc_sh.at[dst_v.at[i]], ssem, add=True)
            return 0

        lax.fori_loop(0, C, body, 0)

        def drain(i, _):
            pltpu.make_async_copy(ones_v, acc_sh.at[dst_v.at[i]], ssem).wait()
            return 0

        lax.fori_loop(0, C, drain, 0)
        plsc.subcore_barrier()
        pltpu.sync_copy(acc_sh.at[pl.ds(s * RP, RP)],
                        out_hbm.at[c, pl.ds(s * RP, RP)])

    return deg_kernel


def _sc_aggregate(C, F, NROWS):
    _bf = jnp.bfloat16

    @functools.partial(
        pl.kernel,
        out_type=jax.ShapeDtypeStruct((NC, N_PAD, F), _bf),
        mesh=_mesh(),
        scratch_types=[
            pltpu.VMEM((C, K), jnp.int32),
            pltpu.VMEM((C, K), jnp.int32),
            pltpu.VMEM((K, F), _bf),
            pltpu.VMEM((ZR, F), _bf),
            pltpu.VMEM_SHARED((N_PAD, F), _bf),
            pltpu.VMEM_SHARED((NROWS, F), _bf),
            pltpu.SemaphoreType.DMA,
        ],
        compiler_params=pltpu.CompilerParams(use_tc_tiling_on_sc=False),
    )
    def agg_kernel(g_hbm, src_hbm, dst_hbm, out_hbm,
                   src_v, dst_v, rows_v, zb_v, acc_sh, g_sh, sem):
        c = lax.axis_index("c")
        s = lax.axis_index("s")
        pltpu.sync_copy(src_hbm.at[c, s], src_v)
        pltpu.sync_copy(dst_hbm.at[c, s], dst_v)
        gr = NROWS // NS
        pltpu.sync_copy(g_hbm.at[pl.ds(s * gr, gr)], g_sh.at[pl.ds(s * gr, gr)])

        def fill_zero(i, _):
            for j in range(F // (2 * L)):
                zb_v[i, pl.ds(j * 2 * L, 2 * L)] = jnp.zeros((2 * L,), _bf)
            return 0

        lax.fori_loop(0, ZR, fill_zero, 0)
        for t in range(RP // ZR):
            pltpu.sync_copy(zb_v, acc_sh.at[pl.ds(s * RP + t * ZR, ZR)])
        plsc.subcore_barrier()

        def body(i, _):
            pltpu.async_copy(g_sh.at[src_v.at[i]], rows_v, sem).wait()
            pltpu.sync_copy(rows_v, acc_sh.at[dst_v.at[i]], add=True)
            return 0

        lax.fori_loop(0, C, body, 0)
        plsc.subcore_barrier()
        pltpu.sync_copy(acc_sh.at[pl.ds(s * RP, RP)],
                        out_hbm.at[c, pl.ds(s * RP, RP)])

    return agg_kernel


def _tc0(x, W1, z_ref):
    z_ref[...] = jnp.dot(x[...], W1[...], preferred_element_type=_f32)


def _tc1(N):
    def body(dp, z, dinv_ref, g1_ref, g1b_ref):
        deg = dp[0, :N, 0:1] + dp[1, :N, 0:1] + 1.0
        dv = lax.rsqrt(deg)
        G = dv * z[...]
        dinv_ref[...] = dv[:, 0]
        g1_ref[...] = G
        g1b_ref[...] = G.astype(jnp.bfloat16)
    return body


def _tc2(N):
    def body(a, g1, dinv, W2, b1, g2_ref, g2b_ref):
        dv = dinv[...].reshape(-1, 1)
        agg = a[0, :N].astype(_f32) + a[1, :N].astype(_f32)
        H1 = jnp.maximum(dv * (agg + g1[...]) + b1[...][None, :], 0.0)
        G = dv * jnp.dot(H1, W2[...], preferred_element_type=_f32)
        g2_ref[...] = G
        g2b_ref[...] = G.astype(jnp.bfloat16)
    return body


def _tc3(N):
    def body(a, g2, dinv, b2, g3_ref, g3b_ref):
        dv = dinv[...].reshape(-1, 1)
        agg = a[0, :N].astype(_f32) + a[1, :N].astype(_f32)
        H2 = jnp.maximum(dv * (agg + g2[...]) + b2[...][None, :], 0.0)
        G = dv * H2
        g3_ref[...] = G
        g3b_ref[...] = G.astype(jnp.bfloat16)
    return body


def _tc4(N):
    def body(a, g3, dinv, W3, b3, out_ref):
        dv = dinv[...].reshape(-1, 1)
        agg = a[0, :N].astype(_f32) + a[1, :N].astype(_f32)
        M = dv * (agg + g3[...])
        O = jnp.dot(M, W3[...], preferred_element_type=_f32) + b3[...][None, :]
        m = jnp.max(O, axis=1, keepdims=True)
        out_ref[...] = O - m - jnp.log(jnp.sum(jnp.exp(O - m), axis=1, keepdims=True))
    return body


def _tc_call(body, shapes_dtypes, *args):
    out_shape = tuple(jax.ShapeDtypeStruct(sh, dt) for sh, dt in shapes_dtypes)
    if len(out_shape) == 1:
        out_shape = out_shape[0]
    return pl.pallas_call(body, out_shape=out_shape)(*args)


def kernel(x, edge_index, W1, b1, W2, b2, W3, b3):
    N = x.shape[0]
    E = edge_index.shape[1]
    H1, H2 = W1.shape[1], W2.shape[1]

    src = edge_index[0].astype(jnp.int32)
    dst = edge_index[1].astype(jnp.int32)
    SL = NC * NS
    C = -(-E // (SL * K))
    pad = SL * C * K - E
    srcp = jnp.concatenate([src, jnp.zeros((pad,), jnp.int32)]).reshape(NC, NS, C, K)
    dstp = jnp.concatenate([dst, jnp.full((pad,), N, jnp.int32)]).reshape(NC, NS, C, K)
    if E % (SL * KD) == 0:
        CD = E // (SL * KD)
        edges5 = edge_index.astype(jnp.int32).reshape(2, NC, NS, CD, KD)
    else:
        CD = C
        edges5 = jnp.stack([srcp, dstp], axis=0)

    bf = jnp.bfloat16
    degp = _sc_degree(CD, edges5.shape[4])(edges5)

    Z1 = _tc_call(_tc0, [((N, H1), _f32)], x, W1)
    dinv, G1, G1b = _tc_call(
        _tc1(N), [((N,), _f32), ((N, H1), _f32), ((N, H1), bf)], degp, Z1)

    a1 = _sc_aggregate(C, H1, N)(G1b, srcp, dstp)
    G2, G2b = _tc_call(
        _tc2(N), [((N, H2), _f32), ((N, H2), bf)], a1, G1, dinv, W2, b1)

    a2 = _sc_aggregate(C, H2, N)(G2b, srcp, dstp)
    G3, G3b = _tc_call(
        _tc3(N), [((N, H2), _f32), ((N, H2), bf)], a2, G2, dinv, b2)

    a3 = _sc_aggregate(C, H2, N)(G3b, srcp, dstp)
    out = _tc_call(
        _tc4(N), [((N, W3.shape[1]), _f32)], a3, G3, dinv, W3, b3)
    return out


# speedup vs baseline: 1.0640x; 1.0002x over previous
import functools

import jax
import jax.numpy as jnp
from jax import lax
from jax.experimental import pallas as pl
from jax.experimental.pallas import tpu as pltpu
from jax.experimental.pallas import tpu_sc as plsc

NC, NS, L = 2, 16, 16
K = 128
KD = 80
N_PAD = 10240
RP = N_PAD // NS
ZR = 64
DEGW = 16

_f32 = jnp.float32


def _mesh():
    return plsc.VectorSubcoreMesh(core_axis_name="c", subcore_axis_name="s")


def _sc_degree(C, K):
    @functools.partial(
        pl.kernel,
        out_type=jax.ShapeDtypeStruct((NC, N_PAD, DEGW), _f32),
        mesh=_mesh(),
        scratch_types=[
            pltpu.VMEM((C, K), jnp.int32),
            pltpu.VMEM((K, DEGW), _f32),
            pltpu.VMEM((ZR, DEGW), _f32),
            pltpu.VMEM_SHARED((N_PAD, DEGW), _f32),
            pltpu.SemaphoreType.DMA,
        ],
        compiler_params=pltpu.CompilerParams(use_tc_tiling_on_sc=False),
    )
    def deg_kernel(edges_hbm, out_hbm, dst_v, ones_v, zb_v, acc_sh, ssem):
        c = lax.axis_index("c")
        s = lax.axis_index("s")
        pltpu.sync_copy(edges_hbm.at[1, c, s], dst_v)

        def fill_ones(i, _):
            ones_v[i, :] = jnp.ones((L,), _f32)
            return 0

        lax.fori_loop(0, K, fill_ones, 0)

        def fill_zero(i, _):
            zb_v[i, :] = jnp.zeros((L,), _f32)
            return 0

        lax.fori_loop(0, ZR, fill_zero, 0)
        for t in range(RP // ZR):
            pltpu.sync_copy(zb_v, acc_sh.at[pl.ds(s * RP + t * ZR, ZR)])
        plsc.subcore_barrier()

        def body(i, _):
            pltpu.async_copy(ones_v, acc_sh.at[dst_v.at[i]], ssem, add=True)
            return 0

        lax.fori_loop(0, C, body, 0)

        def drain(i, _):
            pltpu.make_async_copy(ones_v, acc_sh.at[dst_v.at[i]], ssem).wait()
            return 0

        lax.fori_loop(0, C, drain, 0)
        plsc.subcore_barrier()
        pltpu.sync_copy(acc_sh.at[pl.ds(s * RP, RP)],
                        out_hbm.at[c, pl.ds(s * RP, RP)])

    return deg_kernel


def _sc_aggregate(C, F, NROWS):
    _bf = jnp.bfloat16

    @functools.partial(
        pl.kernel,
        out_type=jax.ShapeDtypeStruct((NC, N_PAD, F), _bf),
        mesh=_mesh(),
        scratch_types=[
            pltpu.VMEM((C, K), jnp.int32),
            pltpu.VMEM((C, K), jnp.int32),
            pltpu.VMEM((K, F), _bf),
            pltpu.VMEM((ZR, F), _bf),
            pltpu.VMEM_SHARED((N_PAD, F), _bf),
            pltpu.VMEM_SHARED((NROWS, F), _bf),
            pltpu.SemaphoreType.DMA,
        ],
        compiler_params=pltpu.CompilerParams(use_tc_tiling_on_sc=False),
    )
    def agg_kernel(g_hbm, src_hbm, dst_hbm, out_hbm,
                   src_v, dst_v, rows_v, zb_v, acc_sh, g_sh, sem):
        c = lax.axis_index("c")
        s = lax.axis_index("s")
        pltpu.sync_copy(src_hbm.at[c, s], src_v)
        pltpu.sync_copy(dst_hbm.at[c, s], dst_v)
        gr = NROWS // NS
        pltpu.sync_copy(g_hbm.at[pl.ds(s * gr, gr)], g_sh.at[pl.ds(s * gr, gr)])

        def fill_zero(i, _):
            for j in range(F // (2 * L)):
                zb_v[i, pl.ds(j * 2 * L, 2 * L)] = jnp.zeros((2 * L,), _bf)
            return 0

        lax.fori_loop(0, ZR, fill_zero, 0)
        for t in range(RP // ZR):
            pltpu.sync_copy(zb_v, acc_sh.at[pl.ds(s * RP + t * ZR, ZR)])
        plsc.subcore_barrier()

        def body(i, _):
            pltpu.async_copy(g_sh.at[src_v.at[i]], rows_v, sem).wait()
            pltpu.sync_copy(rows_v, acc_sh.at[dst_v.at[i]], add=True)
            return 0

        lax.fori_loop(0, C, body, 0)
        plsc.subcore_barrier()
        pltpu.sync_copy(acc_sh.at[pl.ds(s * RP, RP)],
                        out_hbm.at[c, pl.ds(s * RP, RP)])

    return agg_kernel


def _tc0(x, W1, z_ref):
    z_ref[...] = jnp.dot(x[...], W1[...], preferred_element_type=_f32)


def _tc1(N):
    def body(dp, z, dinv_ref, g1_ref, g1b_ref):
        deg = dp[0, :N, 0:1] + dp[1, :N, 0:1] + 1.0
        dv = lax.rsqrt(deg)
        G = dv * z[...]
        dinv_ref[...] = dv[:, 0]
        g1_ref[...] = G
        g1b_ref[...] = G.astype(jnp.bfloat16)
    return body


def _tc2(N):
    def body(a, g1, dinv, W2, b1, g2_ref, g2b_ref):
        dv = dinv[...].reshape(-1, 1)
        agg = a[0, :N].astype(_f32) + a[1, :N].astype(_f32)
        H1 = jnp.maximum(dv * (agg + g1[...]) + b1[...][None, :], 0.0)
        G = dv * jnp.dot(H1, W2[...], preferred_element_type=_f32)
        g2_ref[...] = G
        g2b_ref[...] = G.astype(jnp.bfloat16)
    return body


def _tc3(N):
    def body(a, g2, dinv, b2, g3_ref, g3b_ref):
        dv = dinv[...].reshape(-1, 1)
        agg = a[0, :N].astype(_f32) + a[1, :N].astype(_f32)
        H2 = jnp.maximum(dv * (agg + g2[...]) + b2[...][None, :], 0.0)
        G = dv * H2
        g3_ref[...] = G
        g3b_ref[...] = G.astype(jnp.bfloat16)
    return body


def _tc4(N):
    def body(a, g3, dinv, W3, b3, out_ref):
        dv = dinv[...].reshape(-1, 1)
        agg = a[0, :N].astype(_f32) + a[1, :N].astype(_f32)
        M = dv * (agg + g3[...])
        O = jnp.dot(M, W3[...], preferred_element_type=_f32) + b3[...][None, :]
        m = jnp.max(O, axis=1, keepdims=True)
        out_ref[...] = O - m - jnp.log(jnp.sum(jnp.exp(O - m), axis=1, keepdims=True))
    return body


def _tc_call(body, shapes_dtypes, *args):
    out_shape = tuple(jax.ShapeDtypeStruct(sh, dt) for sh, dt in shapes_dtypes)
    if len(out_shape) == 1:
        out_shape = out_shape[0]
    return pl.pallas_call(body, out_shape=out_shape)(*args)


def kernel(x, edge_index, W1, b1, W2, b2, W3, b3):
    N = x.shape[0]
    E = edge_index.shape[1]
    H1, H2 = W1.shape[1], W2.shape[1]

    src = edge_index[0].astype(jnp.int32)
    dst = edge_index[1].astype(jnp.int32)
    SL = NC * NS
    C = -(-E // (SL * K))
    pad = SL * C * K - E
    srcp = jnp.concatenate([src, jnp.zeros((pad,), jnp.int32)]).reshape(NC, NS, C, K)
    dstp = jnp.concatenate([dst, jnp.full((pad,), N, jnp.int32)]).reshape(NC, NS, C, K)
    if E % (SL * KD) == 0:
        CD = E // (SL * KD)
        edges5 = edge_index.astype(jnp.int32).reshape(2, NC, NS, CD, KD)
    else:
        CD = C
        edges5 = jnp.stack([srcp, dstp], axis=0)

    bf = jnp.bfloat16
    degp = _sc_degree(CD, edges5.shape[4])(edges5)

    Z1 = _tc_call(_tc0, [((N, H1), _f32)], x, W1)
    dinv, G1, G1b = _tc_call(
        _tc1(N), [((N,), _f32), ((N, H1), _f32), ((N, H1), bf)], degp, Z1)

    a1 = _sc_aggregate(C, H1, N)(G1b, srcp, dstp)
    G2, G2b = _tc_call(
        _tc2(N), [((N, H2), _f32), ((N, H2), bf)], a1, G1, dinv, W2, b1)

    a2 = _sc_aggregate(C, H2, N)(G2b, srcp, dstp)
    G3, G3b = _tc_call(
        _tc3(N), [((N, H2), _f32), ((N, H2), bf)], a2, G2, dinv, b2)

    a3 = _sc_aggregate(C, H2, N)(G3b, srcp, dstp)
    out = _tc_call(
        _tc4(N), [((N, W3.shape[1]), _f32)], a3, G3, dinv, W3, b3)
    return out
